# baseline (speedup 1.0000x reference)
import functools

import jax
import jax.numpy as jnp
from jax import lax
from jax.experimental import pallas as pl
from jax.experimental.pallas import tpu as pltpu
from jax.experimental.pallas import tpu_sc as plsc

VOCAB = 1000000
RANK = 32
DIM = 128
BATCH = 4096
HIST = 50

NC = 2
NS = 16
NW = NC * NS

TOTAL = BATCH * HIST
ROWS_PER_TILE = TOTAL // NW
CHUNK = 128
N_CHUNKS = ROWS_PER_TILE // CHUNK

SUPER = 8192
LINES = SUPER // 4
NSUPER = (VOCAB + SUPER - 1) // SUPER
LINES_TOTAL = NSUPER * LINES
VOCAB_VIEW = LINES_TOTAL * 4


def _tc_repack(table_t):
  def body(in_ref, out_ref):
    t = in_ref[...]
    out_ref[...] = jnp.concatenate(
        [t[:, a * LINES:(a + 1) * LINES].T for a in range(4)], axis=1
    ).astype(jnp.bfloat16)

  return pl.pallas_call(
      body,
      grid=(NSUPER,),
      in_specs=[pl.BlockSpec((RANK, SUPER), lambda i: (0, i))],
      out_specs=pl.BlockSpec((LINES, DIM), lambda i: (i, 0)),
      out_shape=jax.ShapeDtypeStruct((LINES_TOTAL, DIM), jnp.bfloat16),
  )(table_t)


def _sc_gather(tab_lin, idx3):
  mesh = plsc.VectorSubcoreMesh(core_axis_name="c", subcore_axis_name="s")

  @functools.partial(
      pl.kernel,
      mesh=mesh,
      compiler_params=pltpu.CompilerParams(use_tc_tiling_on_sc=False),
      out_type=jax.ShapeDtypeStruct((TOTAL, DIM), jnp.bfloat16),
      scratch_types=[
          pltpu.VMEM((N_CHUNKS, CHUNK), jnp.int32),
          pltpu.VMEM((2, CHUNK, RANK), jnp.bfloat16),
          pltpu.SemaphoreType.DMA,
          pltpu.SemaphoreType.DMA,
      ],
  )
  def k(table_hbm, idx_hbm, out_hbm, idx_v, rows_v, gsem, wsem):
    wid = lax.axis_index("s") * NC + lax.axis_index("c")
    base = wid * ROWS_PER_TILE
    pltpu.sync_copy(idx_hbm.at[wid], idx_v)

    def gather_then_write(c):
      slot = lax.rem(c, 2)
      g = pltpu.async_copy(table_hbm.at[idx_v.at[c]], rows_v.at[slot], gsem)
      g.wait()
      pltpu.async_copy(
          rows_v.at[slot],
          out_hbm.at[pl.ds(base + c * CHUNK, CHUNK), pl.ds(0, RANK)],
          wsem,
      )

    def drain_one_write():
      pltpu.make_async_copy(
          rows_v.at[0],
          out_hbm.at[pl.ds(base, CHUNK), pl.ds(0, RANK)],
          wsem,
      ).wait()

    gather_then_write(0)
    gather_then_write(1)

    def body(c, _):
      drain_one_write()
      gather_then_write(c)
      return ()

    lax.fori_loop(2, N_CHUNKS, body, ())
    drain_one_write()
    drain_one_write()

  return k(tab_lin, idx3)


def _tc_project(emb, w_scaled, b_scaled):
  BM = 2048

  def body(emb_ref, w_ref, b_ref, out_ref):
    out_ref[...] = (
        jnp.dot(
            emb_ref[:, :RANK], w_ref[...], preferred_element_type=jnp.float32
        )
        + b_ref[...]
    )

  return pl.pallas_call(
      body,
      grid=(TOTAL // BM,),
      in_specs=[
          pl.BlockSpec((BM, DIM), lambda i: (i, 0)),
          pl.BlockSpec((RANK, DIM), lambda i: (0, 0)),
          pl.BlockSpec((1, DIM), lambda i: (0, 0)),
      ],
      out_specs=pl.BlockSpec((BM, DIM), lambda i: (i, 0)),
      out_shape=jax.ShapeDtypeStruct((TOTAL, DIM), jnp.float32),
  )(emb, w_scaled, b_scaled)


def kernel(x, table, W, b, embed_scale):
  s = embed_scale.astype(table.dtype)
  w_scaled = (W * s).astype(jnp.bfloat16)
  b_scaled = (b * s).astype(jnp.float32).reshape(1, DIM)
  v = jnp.swapaxes(x, 0, 1).reshape(TOTAL).astype(jnp.int32)
  r = v % SUPER
  idxp = (v // SUPER) * SUPER + 4 * (r % LINES) + r // LINES
  idx3 = idxp.reshape(NW, N_CHUNKS, CHUNK)
  tab_lin = _tc_repack(jnp.swapaxes(table, 0, 1))
  emb = _sc_gather(tab_lin.reshape(VOCAB_VIEW, RANK), idx3)
  out = _tc_project(emb, w_scaled, b_scaled)
  return jnp.swapaxes(out.reshape(HIST, BATCH, DIM), 0, 1)


# speedup vs baseline: 3.1334x; 3.1334x over previous
import functools

import jax
import jax.numpy as jnp
from jax import lax
from jax.experimental import pallas as pl
from jax.experimental.pallas import tpu as pltpu
from jax.experimental.pallas import tpu_sc as plsc

VOCAB = 1000000
RANK = 32
DIM = 128
BATCH = 4096
HIST = 50

NC = 2
NS = 16
NW = NC * NS

TOTAL = BATCH * HIST
ROWS_PER_TILE = TOTAL // NW
CHUNK = 128
N_CHUNKS = ROWS_PER_TILE // CHUNK
DEPTH = 6
NBUF = 12

SUPER = 65536
LINES = SUPER // 4
NSUPER = (VOCAB + SUPER - 1) // SUPER
LINES_TOTAL = NSUPER * LINES
VOCAB_VIEW = LINES_TOTAL * 4


def _tc_repack(table_t):
  def body(in_ref, eye_ref, out_ref):
    t = in_ref[...]
    e = eye_ref[...]
    acc = jax.lax.dot_general(
        t[:, 0:LINES], e[0:RANK, :],
        (((0,), (0,)), ((), ())),
        preferred_element_type=jnp.float32,
    )
    for a in range(1, 4):
      acc += jax.lax.dot_general(
          t[:, a * LINES:(a + 1) * LINES], e[a * RANK:(a + 1) * RANK, :],
          (((0,), (0,)), ((), ())),
          preferred_element_type=jnp.float32,
      )
    out_ref[...] = acc

  return pl.pallas_call(
      body,
      grid=(NSUPER,),
      in_specs=[
          pl.BlockSpec((RANK, SUPER), lambda i: (0, i)),
          pl.BlockSpec((DIM, DIM), lambda i: (0, 0)),
      ],
      compiler_params=pltpu.CompilerParams(
          fuse_transposed_lhs_in_matmul=True,
      ),
      out_specs=pl.BlockSpec((LINES, DIM), lambda i: (i, 0)),
      out_shape=jax.ShapeDtypeStruct((LINES_TOTAL, DIM), jnp.float32),
  )(table_t, jnp.eye(DIM, dtype=jnp.float32))


def _sc_gather(tab_lin, idx3):
  mesh = plsc.VectorSubcoreMesh(core_axis_name="c", subcore_axis_name="s")

  @functools.partial(
      pl.kernel,
      mesh=mesh,
      compiler_params=pltpu.CompilerParams(use_tc_tiling_on_sc=False),
      out_type=jax.ShapeDtypeStruct((TOTAL // 4, DIM), jnp.float32),
      scratch_types=[
          pltpu.VMEM((N_CHUNKS, CHUNK), jnp.int32),
          pltpu.VMEM((NBUF, CHUNK, RANK), jnp.float32),
          pltpu.SemaphoreType.DMA,
          pltpu.SemaphoreType.DMA,
      ],
  )
  def k(table_hbm, idx_hbm, out_hbm, idx_v, rows_v, gsem, wsem):
    wid = lax.axis_index("s") * NC + lax.axis_index("c")
    line_base = lax.rem(wid, 8) * ROWS_PER_TILE
    lane_base = lax.div(wid, 8) * RANK
    pltpu.sync_copy(idx_hbm.at[wid], idx_v)

    def fire_gather(c):
      pltpu.async_copy(
          table_hbm.at[idx_v.at[c]], rows_v.at[lax.rem(c, NBUF)], gsem
      )

    def fire_write(c):
      pltpu.async_copy(
          rows_v.at[lax.rem(c, NBUF)],
          out_hbm.at[
              pl.ds(line_base + c * CHUNK, CHUNK), pl.ds(lane_base, RANK)
          ],
          wsem,
      )

    def wait_one(sem):
      pltpu.make_async_copy(
          rows_v.at[0],
          out_hbm.at[pl.ds(line_base, CHUNK), pl.ds(lane_base, RANK)],
          sem,
      ).wait()

    for c in range(DEPTH):
      fire_gather(c)

    def body(c, _):
      wait_one(gsem)
      fire_write(c)

      @pl.when(c >= DEPTH)
      def _():
        wait_one(wsem)

      @pl.when(c + DEPTH < N_CHUNKS)
      def _():
        fire_gather(c + DEPTH)

      return ()

    lax.fori_loop(0, N_CHUNKS, body, ())
    for _ in range(min(DEPTH, N_CHUNKS)):
      wait_one(wsem)

  return k(tab_lin, idx3)


def _tc_project(emb4, w4s, b_scaled):
  BM4 = 12800
  NBLK = TOTAL // 4 // BM4

  def body(emb_ref, w_ref, b_ref, out_ref):
    out_ref[...] = (
        jnp.dot(emb_ref[...], w_ref[0], preferred_element_type=jnp.float32)
        + b_ref[...]
    )

  return pl.pallas_call(
      body,
      grid=(NBLK, 4),
      in_specs=[
          pl.BlockSpec((BM4, DIM), lambda i, a: (i, 0)),
          pl.BlockSpec((1, DIM, DIM), lambda i, a: (a, 0, 0)),
          pl.BlockSpec((1, DIM), lambda i, a: (0, 0)),
      ],
      out_specs=pl.BlockSpec((BM4, DIM), lambda i, a: (a * NBLK + i, 0)),
      out_shape=jax.ShapeDtypeStruct((TOTAL, DIM), jnp.float32),
  )(emb4, w4s, b_scaled)


def kernel(x, table, W, b, embed_scale):
  s = embed_scale.astype(table.dtype)
  w_scaled = (W * s).astype(jnp.float32)
  b_scaled = (b * s).astype(jnp.float32).reshape(1, DIM)
  w4s = jnp.einsum(
      "ab,rd->abrd", jnp.eye(4, dtype=jnp.float32), w_scaled
  ).reshape(4, DIM, DIM)
  v = jnp.swapaxes(x, 0, 1).reshape(TOTAL).astype(jnp.int32)
  r = v % SUPER
  idxp = (v // SUPER) * SUPER + 4 * (r % LINES) + r // LINES
  idx3 = idxp.reshape(NW, N_CHUNKS, CHUNK)
  tab_lin = _tc_repack(jnp.swapaxes(table, 0, 1))
  emb4 = _sc_gather(tab_lin.reshape(VOCAB_VIEW, RANK), idx3)
  out = _tc_project(emb4, w4s, b_scaled)
  return jnp.swapaxes(out.reshape(HIST, BATCH, DIM), 0, 1)
